# Initial kernel scaffold; baseline (speedup 1.0000x reference)
import functools

import jax
import jax.numpy as jnp
from jax import lax
from jax.experimental import pallas as pl
from jax.experimental.pallas import tpu as pltpu
from jax.experimental.pallas import tpu_sc as plsc

NC = 2
NS = 16
NW = NC * NS
CHUNK = 128
LANES = 16


def _sc_spmm(x, src3, dst3, vals3, zeros, n_chunks, n, d):
    rows_per_tile = n // NS
    mesh = plsc.VectorSubcoreMesh(core_axis_name="c", subcore_axis_name="s")

    @functools.partial(
        pl.kernel,
        out_type=jax.ShapeDtypeStruct((NC, n, d), jnp.float32),
        mesh=mesh,
        scratch_types=[
            pltpu.VMEM((n_chunks, CHUNK), jnp.int32),
            pltpu.VMEM((n_chunks, CHUNK), jnp.int32),
            pltpu.VMEM((n_chunks, CHUNK), jnp.float32),
            pltpu.VMEM((CHUNK, d), jnp.float32),
            pltpu.VMEM_SHARED((n, d), jnp.float32),
            pltpu.SemaphoreType.DMA,
        ],
    )
    def k(x_hbm, src_hbm, dst_hbm, vals_hbm, zeros_hbm, out_hbm,
          src_v, dst_v, vals_v, rows_v, acc, sem):
        cid = lax.axis_index("c")
        sid = lax.axis_index("s")
        wid = cid * NS + sid

        pltpu.sync_copy(src_hbm.at[wid], src_v)
        pltpu.sync_copy(dst_hbm.at[wid], dst_v)
        pltpu.sync_copy(vals_hbm.at[wid], vals_v)
        r0 = sid * rows_per_tile
        pltpu.sync_copy(zeros_hbm.at[pl.ds(r0, rows_per_tile)],
                        acc.at[pl.ds(r0, rows_per_tile)])
        plsc.subcore_barrier()

        def chunk_body(c, carry):
            pltpu.async_copy(x_hbm.at[src_v.at[c]], rows_v, sem).wait()

            def scale_body(i, carry2):
                v = vals_v[c, i]
                for j in range(d // LANES):
                    sl = (i, pl.ds(j * LANES, LANES))
                    rows_v[sl] = rows_v[sl] * v
                return carry2

            lax.fori_loop(0, CHUNK, scale_body, 0)
            pltpu.sync_copy(rows_v, acc.at[dst_v.at[c]], add=True)
            return carry

        lax.fori_loop(0, n_chunks, chunk_body, 0)
        plsc.subcore_barrier()
        pltpu.sync_copy(acc.at[pl.ds(r0, rows_per_tile)],
                        out_hbm.at[cid, pl.ds(r0, rows_per_tile)])

    return k(x, src3, dst3, vals3, zeros)


def _epilogue(parts, x, x0, alpha_col, n, d, block):
    def body(parts_ref, x_ref, x0_ref, a_ref, o_ref):
        ax = parts_ref[0] + parts_ref[1]
        alph = jax.nn.sigmoid(a_ref[...])
        o_ref[...] = alph * 0.5 * (ax - x_ref[...]) + x0_ref[...]

    return pl.pallas_call(
        body,
        out_shape=jax.ShapeDtypeStruct((n, d), jnp.float32),
        grid=(n // block,),
        in_specs=[
            pl.BlockSpec((NC, block, d), lambda i: (0, i, 0)),
            pl.BlockSpec((block, d), lambda i: (i, 0)),
            pl.BlockSpec((block, d), lambda i: (i, 0)),
            pl.BlockSpec((block, 1), lambda i: (i, 0)),
        ],
        out_specs=pl.BlockSpec((block, d), lambda i: (i, 0)),
    )(parts, x, x0, alpha_col)


def kernel(t, x, x0, alpha_train, adj_vals, edge_index):
    n, d = x.shape
    e = edge_index.shape[1]
    n_chunks = -(-e // (NW * CHUNK))
    e_pad = NW * CHUNK * n_chunks
    pad = e_pad - e

    dst = jnp.pad(edge_index[0], (0, pad)).reshape(NW, n_chunks, CHUNK)
    src = jnp.pad(edge_index[1], (0, pad)).reshape(NW, n_chunks, CHUNK)
    vals = jnp.pad(adj_vals, (0, pad)).reshape(NW, n_chunks, CHUNK)
    zeros = jnp.zeros((n, d), jnp.float32)

    parts = _sc_spmm(x, src, dst, vals, zeros, n_chunks, n, d)
    return _epilogue(parts, x, x0, alpha_train[:, None], n, d, 2000)


# speedup vs baseline: 4.3796x; 4.3796x over previous
import functools

import jax
import jax.numpy as jnp
from jax import lax
from jax.experimental import pallas as pl
from jax.experimental.pallas import tpu as pltpu
from jax.experimental.pallas import tpu_sc as plsc

NC = 2
NS = 16
NW = NC * NS
CHUNK = 128
LANES = 16


def _sc_spmm(x, src3, dst3, vals3, zeros, n_chunks, n, d):
    rows_per_tile = (n // NS) // 8 * 8
    rem_rows = n - NS * rows_per_tile
    rem_base = NS * rows_per_tile
    mesh = plsc.VectorSubcoreMesh(core_axis_name="c", subcore_axis_name="s")

    @functools.partial(
        pl.kernel,
        out_type=jax.ShapeDtypeStruct((NC, n, d), jnp.float32),
        mesh=mesh,
        scratch_types=[
            pltpu.VMEM((n_chunks, CHUNK), jnp.int32),
            pltpu.VMEM((n_chunks, CHUNK), jnp.int32),
            pltpu.VMEM((n_chunks, CHUNK), jnp.float32),
            pltpu.VMEM((CHUNK, d), jnp.float32),
            pltpu.VMEM_SHARED((n, d), jnp.float32),
            pltpu.SemaphoreType.DMA,
        ],
    )
    def k(x_hbm, src_hbm, dst_hbm, vals_hbm, zeros_hbm, out_hbm,
          src_v, dst_v, vals_v, rows_v, acc, sem):
        cid = lax.axis_index("c")
        sid = lax.axis_index("s")
        wid = cid * NS + sid

        pltpu.sync_copy(src_hbm.at[wid], src_v)
        pltpu.sync_copy(dst_hbm.at[wid], dst_v)
        pltpu.sync_copy(vals_hbm.at[wid], vals_v)
        r0 = sid * rows_per_tile
        pltpu.sync_copy(zeros_hbm.at[pl.ds(r0, rows_per_tile)],
                        acc.at[pl.ds(r0, rows_per_tile)])
        if rem_rows:
            @pl.when(sid == NS - 1)
            def _():
                pltpu.sync_copy(zeros_hbm.at[pl.ds(rem_base, rem_rows)],
                                acc.at[pl.ds(rem_base, rem_rows)])
        plsc.subcore_barrier()

        def chunk_body(c, carry):
            pltpu.async_copy(x_hbm.at[src_v.at[c]], rows_v, sem).wait()

            def scale_body(i16, carry2):
                vvec = vals_v[c, pl.ds(i16 * LANES, LANES)]
                for ii in range(LANES):
                    v = vvec[ii]
                    r = i16 * LANES + ii
                    for j in range(d // LANES):
                        sl = (r, pl.ds(j * LANES, LANES))
                        rows_v[sl] = rows_v[sl] * v
                return carry2

            lax.fori_loop(0, CHUNK // LANES, scale_body, 0)
            pltpu.sync_copy(rows_v, acc.at[dst_v.at[c]], add=True)
            return carry

        lax.fori_loop(0, n_chunks, chunk_body, 0)
        plsc.subcore_barrier()
        pltpu.sync_copy(acc.at[pl.ds(r0, rows_per_tile)],
                        out_hbm.at[cid, pl.ds(r0, rows_per_tile)])
        if rem_rows:
            @pl.when(sid == NS - 1)
            def _():
                pltpu.sync_copy(acc.at[pl.ds(rem_base, rem_rows)],
                                out_hbm.at[cid, pl.ds(rem_base, rem_rows)])

    return k(x, src3, dst3, vals3, zeros)


def _epilogue(parts, x, x0, alpha_col, n, d, block):
    def body(parts_ref, x_ref, x0_ref, a_ref, o_ref):
        ax = parts_ref[0] + parts_ref[1]
        alph = jax.nn.sigmoid(a_ref[...])
        o_ref[...] = alph * 0.5 * (ax - x_ref[...]) + x0_ref[...]

    return pl.pallas_call(
        body,
        out_shape=jax.ShapeDtypeStruct((n, d), jnp.float32),
        grid=(n // block,),
        in_specs=[
            pl.BlockSpec((NC, block, d), lambda i: (0, i, 0)),
            pl.BlockSpec((block, d), lambda i: (i, 0)),
            pl.BlockSpec((block, d), lambda i: (i, 0)),
            pl.BlockSpec((block, 1), lambda i: (i, 0)),
        ],
        out_specs=pl.BlockSpec((block, d), lambda i: (i, 0)),
    )(parts, x, x0, alpha_col)


def kernel(t, x, x0, alpha_train, adj_vals, edge_index):
    n, d = x.shape
    e = edge_index.shape[1]
    n_chunks = -(-e // (NW * CHUNK))
    e_pad = NW * CHUNK * n_chunks
    pad = e_pad - e

    dst = jnp.pad(edge_index[0], (0, pad)).reshape(NW, n_chunks, CHUNK)
    src = jnp.pad(edge_index[1], (0, pad)).reshape(NW, n_chunks, CHUNK)
    vals = jnp.pad(adj_vals, (0, pad)).reshape(NW, n_chunks, CHUNK)
    zeros = jnp.zeros((n, d), jnp.float32)

    parts = _sc_spmm(x, src, dst, vals, zeros, n_chunks, n, d)
    return _epilogue(parts, x, x0, alpha_train[:, None], n, d, 2000)


# speedup vs baseline: 6.4679x; 1.4768x over previous
import functools

import jax
import jax.numpy as jnp
from jax import lax
from jax.experimental import pallas as pl
from jax.experimental.pallas import tpu as pltpu
from jax.experimental.pallas import tpu_sc as plsc

NC = 2
NS = 16
NW = NC * NS
CHUNK = 112
LANES = 16
NROW = 3
NEB = 6


def _sc_spmm(x, epack, evals, zeros, n_chunks, n, d):
    rows_per_tile = (n // NS) // 8 * 8
    rem_rows = n - NS * rows_per_tile
    rem_base = NS * rows_per_tile
    mesh = plsc.VectorSubcoreMesh(core_axis_name="c", subcore_axis_name="s")

    assert n_chunks % NEB == 0 and n_chunks >= 2 * NEB
    n_groups = n_chunks // NEB

    @functools.partial(
        pl.kernel,
        out_type=jax.ShapeDtypeStruct((NC, n, d), jnp.float32),
        mesh=mesh,
        scratch_types=[
            pltpu.VMEM((NEB, 2, CHUNK), jnp.int32),
            pltpu.VMEM((NEB, CHUNK), jnp.float32),
            pltpu.VMEM((NROW, CHUNK, d), jnp.float32),
            pltpu.VMEM_SHARED((n, d), jnp.float32),
            pltpu.SemaphoreType.DMA((NEB,)),
            pltpu.SemaphoreType.DMA((NEB,)),
            pltpu.SemaphoreType.DMA((NROW,)),
            pltpu.SemaphoreType.DMA((NROW,)),
        ],
    )
    def k(x_hbm, ep_hbm, ev_hbm, zeros_hbm, out_hbm,
          eb_v, vb_v, rows_v, acc, esem, vsem, gsem, ssem):
        cid = lax.axis_index("c")
        sid = lax.axis_index("s")
        wid = cid * NS + sid

        r0 = sid * rows_per_tile
        pltpu.sync_copy(zeros_hbm.at[pl.ds(r0, rows_per_tile)],
                        acc.at[pl.ds(r0, rows_per_tile)])
        if rem_rows:
            @pl.when(sid == NS - 1)
            def _():
                pltpu.sync_copy(zeros_hbm.at[pl.ds(rem_base, rem_rows)],
                                acc.at[pl.ds(rem_base, rem_rows)])
        plsc.subcore_barrier()

        def eb_start(c, s):
            m = s % NEB
            pltpu.async_copy(ep_hbm.at[wid, c], eb_v.at[m], esem.at[m])
            pltpu.async_copy(ev_hbm.at[wid, c], vb_v.at[m], vsem.at[m])

        def eb_wait(c, s):
            m = s % NEB
            pltpu.make_async_copy(ep_hbm.at[wid, c], eb_v.at[m],
                                  esem.at[m]).wait()
            pltpu.make_async_copy(ev_hbm.at[wid, c], vb_v.at[m],
                                  vsem.at[m]).wait()

        def gather_start(c, s):
            m, r = s % NEB, s % NROW
            pltpu.async_copy(x_hbm.at[eb_v.at[m, 0]], rows_v.at[r],
                             gsem.at[r])

        def gather_wait(c, s):
            m, r = s % NEB, s % NROW
            pltpu.make_async_copy(x_hbm.at[eb_v.at[m, 0]], rows_v.at[r],
                                  gsem.at[r]).wait()

        def scat_start(c, s):
            m, r = s % NEB, s % NROW
            pltpu.async_copy(rows_v.at[r], acc.at[eb_v.at[m, 1]],
                             ssem.at[r], add=True)

        def scat_wait(c, s):
            m, r = s % NEB, s % NROW
            pltpu.make_async_copy(rows_v.at[r], acc.at[eb_v.at[m, 1]],
                                  ssem.at[r]).wait()

        def scale(s):
            m, r = s % NEB, s % NROW

            def scale_body(i16, carry2):
                vvec = vb_v[m, pl.ds(i16 * LANES, LANES)]
                for ii in range(LANES):
                    v = vvec[ii]
                    row = i16 * LANES + ii
                    for j in range(d // LANES):
                        sl = (r, row, pl.ds(j * LANES, LANES))
                        rows_v[sl] = rows_v[sl] * v
                return carry2

            lax.fori_loop(0, CHUNK // LANES, scale_body, 0)

        def steady(c, s, first=False, do_g2=True, do_e4=True):
            gather_wait(c, s)
            scale(s)
            scat_start(c, s)
            if not first:
                scat_wait(c - 1, s - 1)
            if do_g2:
                eb_wait(c + 2, s + 2)
                gather_start(c + 2, s + 2)
            if do_e4:
                eb_start(c + 4, s + 4)

        for c in range(4):
            eb_start(c, c)
        for c in range(2):
            eb_wait(c, c)
            gather_start(c, c)
        steady(0, 0, first=True)
        for c in range(1, NEB):
            steady(c, c)

        def group_body(g, carry):
            c0 = g * NEB
            for kk in range(NEB):
                steady(c0 + kk, kk)
            return carry

        lax.fori_loop(1, n_groups - 1, group_body, 0)

        cl = n_chunks - NEB
        for kk in range(NEB):
            steady(cl + kk, kk, do_g2=(kk < NEB - 2), do_e4=(kk < NEB - 4))
        scat_wait(n_chunks - 1, NEB - 1)
        plsc.subcore_barrier()

        pltpu.sync_copy(acc.at[pl.ds(r0, rows_per_tile)],
                        out_hbm.at[cid, pl.ds(r0, rows_per_tile)])
        if rem_rows:
            @pl.when(sid == NS - 1)
            def _():
                pltpu.sync_copy(acc.at[pl.ds(rem_base, rem_rows)],
                                out_hbm.at[cid, pl.ds(rem_base, rem_rows)])

    return k(x, epack, evals, zeros)


def _epilogue(parts, x, x0, alpha_col, n, d, block):
    def body(parts_ref, x_ref, x0_ref, a_ref, o_ref):
        ax = parts_ref[0] + parts_ref[1]
        alph = jax.nn.sigmoid(a_ref[...])
        o_ref[...] = alph * 0.5 * (ax - x_ref[...]) + x0_ref[...]

    return pl.pallas_call(
        body,
        out_shape=jax.ShapeDtypeStruct((n, d), jnp.float32),
        grid=(n // block,),
        in_specs=[
            pl.BlockSpec((NC, block, d), lambda i: (0, i, 0)),
            pl.BlockSpec((block, d), lambda i: (i, 0)),
            pl.BlockSpec((block, d), lambda i: (i, 0)),
            pl.BlockSpec((block, 1), lambda i: (i, 0)),
        ],
        out_specs=pl.BlockSpec((block, d), lambda i: (i, 0)),
    )(parts, x, x0, alpha_col)


def kernel(t, x, x0, alpha_train, adj_vals, edge_index):
    n, d = x.shape
    e = edge_index.shape[1]
    n_chunks = -(-e // (NW * CHUNK))
    n_chunks = -(-n_chunks // NEB) * NEB
    e_pad = NW * CHUNK * n_chunks
    pad = e_pad - e

    dst = jnp.pad(edge_index[0], (0, pad)).reshape(NW, n_chunks, CHUNK)
    src = jnp.pad(edge_index[1], (0, pad)).reshape(NW, n_chunks, CHUNK)
    evals = jnp.pad(adj_vals, (0, pad)).reshape(NW, n_chunks, CHUNK)
    epack = jnp.stack([src, dst], axis=2)
    zeros = jnp.zeros((n, d), jnp.float32)

    parts = _sc_spmm(x, epack, evals, zeros, n_chunks, n, d)
    return _epilogue(parts, x, x0, alpha_train[:, None], n, d, 2000)


# speedup vs baseline: 11.1426x; 1.7228x over previous
import functools

import jax
import jax.numpy as jnp
from jax import lax
from jax.experimental import pallas as pl
from jax.experimental.pallas import tpu as pltpu
from jax.experimental.pallas import tpu_sc as plsc

NC = 2
NS = 16
NW = NC * NS
CHUNK = 112
LANES = 16
NROW = 3
NEB = 6


def _sc_spmm(x, epack, evals, zeros, n_chunks, n, d):
    rows_per_tile = (n // NS) // 8 * 8
    rem_rows = n - NS * rows_per_tile
    rem_base = NS * rows_per_tile
    mesh = plsc.VectorSubcoreMesh(core_axis_name="c", subcore_axis_name="s")

    assert n_chunks % NEB == 0 and n_chunks >= 2 * NEB
    n_groups = n_chunks // NEB

    @functools.partial(
        pl.kernel,
        out_type=jax.ShapeDtypeStruct((NC, n, d), jnp.float32),
        mesh=mesh,
        scratch_types=[
            pltpu.VMEM((NEB, 2, CHUNK), jnp.int32),
            pltpu.VMEM((NEB, CHUNK), jnp.float32),
            pltpu.VMEM((NROW, CHUNK, d), jnp.float32),
            pltpu.VMEM_SHARED((n, d), jnp.float32),
            pltpu.SemaphoreType.DMA((NEB,)),
            pltpu.SemaphoreType.DMA((NEB,)),
            pltpu.SemaphoreType.DMA((NROW,)),
            pltpu.SemaphoreType.DMA((NROW,)),
        ],
    )
    def k(x_hbm, ep_hbm, ev_hbm, zeros_hbm, out_hbm,
          eb_v, vb_v, rows_v, acc, esem, vsem, gsem, ssem):
        cid = lax.axis_index("c")
        sid = lax.axis_index("s")
        wid = cid * NS + sid

        r0 = sid * rows_per_tile
        pltpu.sync_copy(zeros_hbm.at[pl.ds(r0, rows_per_tile)],
                        acc.at[pl.ds(r0, rows_per_tile)])
        if rem_rows:
            @pl.when(sid == NS - 1)
            def _():
                pltpu.sync_copy(zeros_hbm.at[pl.ds(rem_base, rem_rows)],
                                acc.at[pl.ds(rem_base, rem_rows)])
        plsc.subcore_barrier()

        def eb_start(c, s):
            m = s % NEB
            pltpu.async_copy(ep_hbm.at[wid, c], eb_v.at[m], esem.at[m])
            pltpu.async_copy(ev_hbm.at[wid, c], vb_v.at[m], vsem.at[m])

        def eb_wait(c, s):
            m = s % NEB
            pltpu.make_async_copy(ep_hbm.at[wid, c], eb_v.at[m],
                                  esem.at[m]).wait()
            pltpu.make_async_copy(ev_hbm.at[wid, c], vb_v.at[m],
                                  vsem.at[m]).wait()

        def gather_start(c, s):
            m, r = s % NEB, s % NROW
            pltpu.async_copy(x_hbm.at[eb_v.at[m, 0]], rows_v.at[r],
                             gsem.at[r])

        def gather_wait(c, s):
            m, r = s % NEB, s % NROW
            pltpu.make_async_copy(x_hbm.at[eb_v.at[m, 0]], rows_v.at[r],
                                  gsem.at[r]).wait()

        def scat_start(c, s):
            m, r = s % NEB, s % NROW
            pltpu.async_copy(rows_v.at[r], acc.at[eb_v.at[m, 1]],
                             ssem.at[r], add=True)

        def scat_wait(c, s):
            m, r = s % NEB, s % NROW
            pltpu.make_async_copy(rows_v.at[r], acc.at[eb_v.at[m, 1]],
                                  ssem.at[r]).wait()

        def scale(s):
            m, r = s % NEB, s % NROW

            def scale_body(i16, carry2):
                vvec = vb_v[m, pl.ds(i16 * LANES, LANES)]
                for ii in range(LANES):
                    v = vvec[ii]
                    row = i16 * LANES + ii
                    for j in range(d // LANES):
                        sl = (r, row, pl.ds(j * LANES, LANES))
                        rows_v[sl] = rows_v[sl] * v
                return carry2

            lax.fori_loop(0, CHUNK // LANES, scale_body, 0)

        def steady(c, s, first=False, do_g2=True, do_e4=True):
            gather_wait(c, s)
            scale(s)
            scat_start(c, s)
            if not first:
                scat_wait(c - 1, s - 1)
            if do_g2:
                eb_wait(c + 2, s + 2)
                gather_start(c + 2, s + 2)
            if do_e4:
                eb_start(c + 4, s + 4)

        for c in range(4):
            eb_start(c, c)
        for c in range(2):
            eb_wait(c, c)
            gather_start(c, c)
        steady(0, 0, first=True)
        for c in range(1, NEB):
            steady(c, c)

        def group_body(g, carry):
            c0 = g * NEB
            for kk in range(NEB):
                steady(c0 + kk, kk)
            return carry

        lax.fori_loop(1, n_groups - 1, group_body, 0)

        cl = n_chunks - NEB
        for kk in range(NEB):
            steady(cl + kk, kk, do_g2=(kk < NEB - 2), do_e4=(kk < NEB - 4))
        scat_wait(n_chunks - 1, NEB - 1)
        plsc.subcore_barrier()

        pltpu.sync_copy(acc.at[pl.ds(r0, rows_per_tile)],
                        out_hbm.at[cid, pl.ds(r0, rows_per_tile)])
        if rem_rows:
            @pl.when(sid == NS - 1)
            def _():
                pltpu.sync_copy(acc.at[pl.ds(rem_base, rem_rows)],
                                out_hbm.at[cid, pl.ds(rem_base, rem_rows)])

    return k(x, epack, evals, zeros)


def _epilogue(parts, x, x0, alpha_col, n, d, block):
    def body(parts_ref, x_ref, x0_ref, a_ref, o_ref):
        ax = parts_ref[0] + parts_ref[1]
        alph = jax.nn.sigmoid(a_ref[...])
        o_ref[...] = alph * 0.5 * (ax - x_ref[...]) + x0_ref[...]

    return pl.pallas_call(
        body,
        out_shape=jax.ShapeDtypeStruct((n, d), jnp.float32),
        grid=(n // block,),
        in_specs=[
            pl.BlockSpec((NC, block, d), lambda i: (0, i, 0)),
            pl.BlockSpec((block, d), lambda i: (i, 0)),
            pl.BlockSpec((block, d), lambda i: (i, 0)),
            pl.BlockSpec((block, 1), lambda i: (i, 0)),
        ],
        out_specs=pl.BlockSpec((block, d), lambda i: (i, 0)),
    )(parts, x, x0, alpha_col)


def kernel(t, x, x0, alpha_train, adj_vals, edge_index):
    n, d = x.shape
    e = edge_index.shape[1]
    n_chunks = -(-e // (NW * CHUNK))
    n_chunks = -(-n_chunks // NEB) * NEB
    e_pad = NW * CHUNK * n_chunks
    pad = e_pad - e

    spread = jnp.arange(pad, dtype=edge_index.dtype) % n
    dst = jnp.concatenate([edge_index[0], spread]).reshape(NW, n_chunks, CHUNK)
    src = jnp.concatenate([edge_index[1], spread]).reshape(NW, n_chunks, CHUNK)
    evals = jnp.pad(adj_vals, (0, pad)).reshape(NW, n_chunks, CHUNK)
    epack = jnp.stack([src, dst], axis=2)
    zeros = jnp.zeros((n, d), jnp.float32)

    parts = _sc_spmm(x, epack, evals, zeros, n_chunks, n, d)
    return _epilogue(parts, x, x0, alpha_train[:, None], n, d, 2000)


# speedup vs baseline: 11.9734x; 1.0746x over previous
import functools

import jax
import jax.numpy as jnp
from jax import lax
from jax.experimental import pallas as pl
from jax.experimental.pallas import tpu as pltpu
from jax.experimental.pallas import tpu_sc as plsc

NC = 2
NS = 16
NW = NC * NS
CHUNK = 112
LANES = 16
NROW = 3
NEB = 6


def _sc_spmm(x, srcp, dstp, evp, n_chunks, n, d):
    rows_per_tile = (n // NS) // 8 * 8
    rem_rows = n - NS * rows_per_tile
    rem_base = NS * rows_per_tile
    n_zfull, n_zrem = divmod(rows_per_tile, CHUNK)
    mesh = plsc.VectorSubcoreMesh(core_axis_name="c", subcore_axis_name="s")

    assert n_chunks % NEB == 0 and n_chunks >= 3 * NEB
    n_groups = n_chunks // NEB - 1
    n_tail = NEB

    @functools.partial(
        pl.kernel,
        out_type=jax.ShapeDtypeStruct((NC, n, d), jnp.float32),
        mesh=mesh,
        scratch_types=[
            pltpu.VMEM((NEB, CHUNK), jnp.int32),
            pltpu.VMEM((NEB, CHUNK), jnp.int32),
            pltpu.VMEM((NEB, CHUNK), jnp.float32),
            pltpu.VMEM((NROW, CHUNK, d), jnp.float32),
            pltpu.VMEM_SHARED((n, d), jnp.float32),
            pltpu.SemaphoreType.DMA((NEB,)),
            pltpu.SemaphoreType.DMA((NEB,)),
            pltpu.SemaphoreType.DMA((NEB,)),
            pltpu.SemaphoreType.DMA((NROW,)),
            pltpu.SemaphoreType.DMA((NROW,)),
        ],
    )
    def k(x_hbm, sp_hbm, dp_hbm, ev_hbm, out_hbm,
          sb_v, db_v, vb_v, rows_v, acc, s_sem, d_sem, v_sem, gsem, ssem):
        cid = lax.axis_index("c")
        sid = lax.axis_index("s")
        wid = cid * NS + sid

        zvec = jnp.zeros((LANES,), jnp.float32)

        def zero_body(i, carry):
            for j in range(d // LANES):
                rows_v[0, i, pl.ds(j * LANES, LANES)] = zvec
            return carry

        lax.fori_loop(0, CHUNK, zero_body, 0)
        r0 = sid * rows_per_tile
        for q in range(n_zfull):
            pltpu.sync_copy(rows_v.at[0],
                            acc.at[pl.ds(r0 + q * CHUNK, CHUNK)])
        if n_zrem:
            pltpu.sync_copy(rows_v.at[0, pl.ds(0, n_zrem)],
                            acc.at[pl.ds(r0 + n_zfull * CHUNK, n_zrem)])
        if rem_rows:
            @pl.when(sid == NS - 1)
            def _():
                pltpu.sync_copy(rows_v.at[0, pl.ds(0, rem_rows)],
                                acc.at[pl.ds(rem_base, rem_rows)])
        plsc.subcore_barrier()

        def eb_start(c, s):
            m = s % NEB
            pltpu.async_copy(sp_hbm.at[wid, c], sb_v.at[m], s_sem.at[m])
            pltpu.async_copy(dp_hbm.at[wid, c], db_v.at[m], d_sem.at[m])
            pltpu.async_copy(ev_hbm.at[wid, c], vb_v.at[m], v_sem.at[m])

        def eb_wait(c, s):
            m = s % NEB
            pltpu.make_async_copy(sp_hbm.at[wid, c], sb_v.at[m],
                                  s_sem.at[m]).wait()
            pltpu.make_async_copy(dp_hbm.at[wid, c], db_v.at[m],
                                  d_sem.at[m]).wait()
            pltpu.make_async_copy(ev_hbm.at[wid, c], vb_v.at[m],
                                  v_sem.at[m]).wait()

        def gather_start(c, s):
            m, r = s % NEB, s % NROW
            pltpu.async_copy(x_hbm.at[sb_v.at[m]], rows_v.at[r], gsem.at[r])

        def gather_wait(c, s):
            m, r = s % NEB, s % NROW
            pltpu.make_async_copy(x_hbm.at[sb_v.at[m]], rows_v.at[r],
                                  gsem.at[r]).wait()

        def scat_start(c, s):
            m, r = s % NEB, s % NROW
            pltpu.async_copy(rows_v.at[r], acc.at[db_v.at[m]], ssem.at[r],
                             add=True)

        def scat_wait(c, s):
            m, r = s % NEB, s % NROW
            pltpu.make_async_copy(rows_v.at[r], acc.at[db_v.at[m]],
                                  ssem.at[r]).wait()

        def scale(s):
            m, r = s % NEB, s % NROW

            def scale_body(i16, carry2):
                vvec = vb_v[m, pl.ds(i16 * LANES, LANES)]
                for ii in range(LANES):
                    v = vvec[ii]
                    row = i16 * LANES + ii
                    for j in range(d // LANES):
                        sl = (r, row, pl.ds(j * LANES, LANES))
                        rows_v[sl] = rows_v[sl] * v
                return carry2

            lax.fori_loop(0, CHUNK // LANES, scale_body, 0)

        def steady(c, s, first=False, do_g2=True, do_e4=True):
            gather_wait(c, s)
            scale(s)
            scat_start(c, s)
            if not first:
                scat_wait(c - 1, s - 1)
            if do_g2:
                eb_wait(c + 2, s + 2)
                gather_start(c + 2, s + 2)
            if do_e4:
                eb_start(c + 4, s + 4)

        for c in range(4):
            eb_start(c, c)
        for c in range(2):
            eb_wait(c, c)
            gather_start(c, c)
        steady(0, 0, first=True)
        for c in range(1, NEB):
            steady(c, c)

        def group_body(g, carry):
            c0 = g * NEB
            for kk in range(NEB):
                steady(c0 + kk, kk)
            return carry

        lax.fori_loop(1, n_groups, group_body, 0)

        cl = n_groups * NEB
        for kk in range(n_tail):
            steady(cl + kk, kk, do_g2=(kk < n_tail - 2),
                   do_e4=(kk < n_tail - 4))
        scat_wait(n_chunks - 1, n_tail - 1)
        plsc.subcore_barrier()

        pltpu.sync_copy(acc.at[pl.ds(r0, rows_per_tile)],
                        out_hbm.at[cid, pl.ds(r0, rows_per_tile)])
        if rem_rows:
            @pl.when(sid == NS - 1)
            def _():
                pltpu.sync_copy(acc.at[pl.ds(rem_base, rem_rows)],
                                out_hbm.at[cid, pl.ds(rem_base, rem_rows)])

    return k(x, srcp, dstp, evp)


def _epilogue(parts, x, x0, alpha_col, n, d, block):
    def body(parts_ref, x_ref, x0_ref, a_ref, o_ref):
        ax = parts_ref[0] + parts_ref[1]
        alph = jax.nn.sigmoid(a_ref[...])
        o_ref[...] = alph * 0.5 * (ax - x_ref[...]) + x0_ref[...]

    return pl.pallas_call(
        body,
        out_shape=jax.ShapeDtypeStruct((n, d), jnp.float32),
        grid=(n // block,),
        in_specs=[
            pl.BlockSpec((NC, block, d), lambda i: (0, i, 0)),
            pl.BlockSpec((block, d), lambda i: (i, 0)),
            pl.BlockSpec((block, d), lambda i: (i, 0)),
            pl.BlockSpec((block, 1), lambda i: (i, 0)),
        ],
        out_specs=pl.BlockSpec((block, d), lambda i: (i, 0)),
    )(parts, x, x0, alpha_col)


def kernel(t, x, x0, alpha_train, adj_vals, edge_index):
    n, d = x.shape
    e = edge_index.shape[1]
    n_chunks = -(-e // (NW * CHUNK))
    n_chunks = -(-n_chunks // NEB) * NEB
    e_pad = NW * CHUNK * n_chunks
    pad = e_pad - e

    spread = jnp.arange(pad, dtype=edge_index.dtype) % n
    dstp = jnp.concatenate([edge_index[0], spread]).reshape(NW, n_chunks,
                                                            CHUNK)
    srcp = jnp.concatenate([edge_index[1], spread]).reshape(NW, n_chunks,
                                                            CHUNK)
    evp = jnp.pad(adj_vals, (0, pad)).reshape(NW, n_chunks, CHUNK)

    parts = _sc_spmm(x, srcp, dstp, evp, n_chunks, n, d)
    return _epilogue(parts, x, x0, alpha_train[:, None], n, d, 2000)


# speedup vs baseline: 12.5948x; 1.0519x over previous
import functools

import jax
import jax.numpy as jnp
from jax import lax
from jax.experimental import pallas as pl
from jax.experimental.pallas import tpu as pltpu
from jax.experimental.pallas import tpu_sc as plsc

NC = 2
NS = 16
NW = NC * NS
CHUNK = 112
LANES = 16
NROW = 3
NEB = 6


def _sc_spmm(x, epk, evp, n_chunks, n, d):
    rows_per_tile = (n // NS) // 8 * 8
    rem_rows = n - NS * rows_per_tile
    rem_base = NS * rows_per_tile
    n_zfull, n_zrem = divmod(rows_per_tile, CHUNK)
    mesh = plsc.VectorSubcoreMesh(core_axis_name="c", subcore_axis_name="s")

    assert n_chunks % NEB == 0 and n_chunks >= 3 * NEB
    n_groups = n_chunks // NEB - 1
    n_tail = NEB

    @functools.partial(
        pl.kernel,
        out_type=jax.ShapeDtypeStruct((NC, n, d), jnp.float32),
        mesh=mesh,
        scratch_types=[
            pltpu.VMEM((NEB, CHUNK), jnp.int32),
            pltpu.VMEM((NEB, CHUNK), jnp.int32),
            pltpu.VMEM((NEB, CHUNK), jnp.float32),
            pltpu.VMEM((NROW, CHUNK, d), jnp.float32),
            pltpu.VMEM_SHARED((n, d), jnp.float32),
            pltpu.SemaphoreType.DMA((NEB,)),
            pltpu.SemaphoreType.DMA((NEB,)),
            pltpu.SemaphoreType.DMA((NEB,)),
            pltpu.SemaphoreType.DMA((NROW,)),
            pltpu.SemaphoreType.DMA((NROW,)),
        ],
    )
    def k(x_hbm, ep_hbm, ev_hbm, out_hbm,
          sb_v, db_v, vb_v, rows_v, acc, s_sem, d_sem, v_sem, gsem, ssem):
        cid = lax.axis_index("c")
        sid = lax.axis_index("s")
        wid = cid * NS + sid

        zvec = jnp.zeros((LANES,), jnp.float32)

        def zero_body(i, carry):
            for j in range(d // LANES):
                rows_v[0, i, pl.ds(j * LANES, LANES)] = zvec
            return carry

        lax.fori_loop(0, CHUNK, zero_body, 0)
        r0 = sid * rows_per_tile
        for q in range(n_zfull):
            pltpu.sync_copy(rows_v.at[0],
                            acc.at[pl.ds(r0 + q * CHUNK, CHUNK)])
        if n_zrem:
            pltpu.sync_copy(rows_v.at[0, pl.ds(0, n_zrem)],
                            acc.at[pl.ds(r0 + n_zfull * CHUNK, n_zrem)])
        if rem_rows:
            @pl.when(sid == NS - 1)
            def _():
                pltpu.sync_copy(rows_v.at[0, pl.ds(0, rem_rows)],
                                acc.at[pl.ds(rem_base, rem_rows)])
        plsc.subcore_barrier()

        def eb_start(c, s):
            m = s % NEB
            pltpu.async_copy(ep_hbm.at[1, wid, c], sb_v.at[m], s_sem.at[m])
            pltpu.async_copy(ep_hbm.at[0, wid, c], db_v.at[m], d_sem.at[m])
            pltpu.async_copy(ev_hbm.at[wid, c], vb_v.at[m], v_sem.at[m])

        def eb_wait(c, s):
            m = s % NEB
            pltpu.make_async_copy(ep_hbm.at[1, wid, c], sb_v.at[m],
                                  s_sem.at[m]).wait()
            pltpu.make_async_copy(ep_hbm.at[0, wid, c], db_v.at[m],
                                  d_sem.at[m]).wait()
            pltpu.make_async_copy(ev_hbm.at[wid, c], vb_v.at[m],
                                  v_sem.at[m]).wait()

        def gather_start(c, s):
            m, r = s % NEB, s % NROW
            pltpu.async_copy(x_hbm.at[sb_v.at[m]], rows_v.at[r], gsem.at[r])

        def gather_wait(c, s):
            m, r = s % NEB, s % NROW
            pltpu.make_async_copy(x_hbm.at[sb_v.at[m]], rows_v.at[r],
                                  gsem.at[r]).wait()

        def scat_start(c, s):
            m, r = s % NEB, s % NROW
            pltpu.async_copy(rows_v.at[r], acc.at[db_v.at[m]], ssem.at[r],
                             add=True)

        def scat_wait(c, s):
            m, r = s % NEB, s % NROW
            pltpu.make_async_copy(rows_v.at[r], acc.at[db_v.at[m]],
                                  ssem.at[r]).wait()

        def scale(s):
            m, r = s % NEB, s % NROW

            def scale_body(i16, carry2):
                vvec = vb_v[m, pl.ds(i16 * LANES, LANES)]
                for ii in range(LANES):
                    v = vvec[ii]
                    row = i16 * LANES + ii
                    for j in range(d // LANES):
                        sl = (r, row, pl.ds(j * LANES, LANES))
                        rows_v[sl] = rows_v[sl] * v
                return carry2

            lax.fori_loop(0, CHUNK // LANES, scale_body, 0)

        def steady(c, s, first=False, do_g2=True, do_e4=True):
            gather_wait(c, s)
            scale(s)
            scat_start(c, s)
            if not first:
                scat_wait(c - 1, s - 1)
            if do_g2:
                eb_wait(c + 2, s + 2)
                gather_start(c + 2, s + 2)
            if do_e4:
                eb_start(c + 4, s + 4)

        for c in range(4):
            eb_start(c, c)
        for c in range(2):
            eb_wait(c, c)
            gather_start(c, c)
        steady(0, 0, first=True)
        for c in range(1, NEB):
            steady(c, c)

        def group_body(g, carry):
            c0 = g * NEB
            for kk in range(NEB):
                steady(c0 + kk, kk)
            return carry

        lax.fori_loop(1, n_groups, group_body, 0)

        cl = n_groups * NEB
        for kk in range(n_tail):
            steady(cl + kk, kk, do_g2=(kk < n_tail - 2),
                   do_e4=(kk < n_tail - 4))
        scat_wait(n_chunks - 1, n_tail - 1)
        plsc.subcore_barrier()

        pltpu.sync_copy(acc.at[pl.ds(r0, rows_per_tile)],
                        out_hbm.at[cid, pl.ds(r0, rows_per_tile)])
        if rem_rows:
            @pl.when(sid == NS - 1)
            def _():
                pltpu.sync_copy(acc.at[pl.ds(rem_base, rem_rows)],
                                out_hbm.at[cid, pl.ds(rem_base, rem_rows)])

    return k(x, epk, evp)


def _epilogue(parts, x, x0, alpha_col, n, d, block):
    def body(parts_ref, x_ref, x0_ref, a_ref, o_ref):
        ax = parts_ref[0] + parts_ref[1]
        alph = jax.nn.sigmoid(a_ref[...].astype(jnp.float32))
        o_ref[...] = alph * 0.5 * (ax - x_ref[...]) + x0_ref[...]

    return pl.pallas_call(
        body,
        out_shape=jax.ShapeDtypeStruct((n, d), jnp.float32),
        grid=(n // block,),
        in_specs=[
            pl.BlockSpec((NC, block, d), lambda i: (0, i, 0)),
            pl.BlockSpec((block, d), lambda i: (i, 0)),
            pl.BlockSpec((block, d), lambda i: (i, 0)),
            pl.BlockSpec((block, 1), lambda i: (i, 0)),
        ],
        out_specs=pl.BlockSpec((block, d), lambda i: (i, 0)),
    )(parts, x, x0, alpha_col)


def kernel(t, x, x0, alpha_train, adj_vals, edge_index):
    n, d = x.shape
    e = edge_index.shape[1]
    n_chunks = -(-e // (NW * CHUNK))
    n_chunks = -(-n_chunks // NEB) * NEB
    e_pad = NW * CHUNK * n_chunks
    pad = e_pad - e

    spread = jnp.arange(pad, dtype=edge_index.dtype) % n
    epk = jnp.concatenate(
        [edge_index, jnp.broadcast_to(spread, (2, pad))],
        axis=1).reshape(2, NW, n_chunks, CHUNK)
    evp = jnp.pad(adj_vals, (0, pad)).reshape(NW, n_chunks, CHUNK)

    parts = _sc_spmm(x, epk, evp, n_chunks, n, d)
    alpha_col = alpha_train.astype(jnp.bfloat16)[:, None]
    return _epilogue(parts, x, x0, alpha_col, n, d, 2000)


# speedup vs baseline: 12.7821x; 1.0149x over previous
import functools

import jax
import jax.numpy as jnp
from jax import lax
from jax.experimental import pallas as pl
from jax.experimental.pallas import tpu as pltpu
from jax.experimental.pallas import tpu_sc as plsc

NC = 2
NS = 16
NW = NC * NS
CHUNK = 112
LANES = 16
NROW = 3
NEB = 6
GROUP = 6


def _sc_spmm(x, epk, evp, n_chunks, n, d):
    rows_per_tile = (n // NS) // 8 * 8
    rem_rows = n - NS * rows_per_tile
    rem_base = NS * rows_per_tile
    n_zfull, n_zrem = divmod(rows_per_tile, CHUNK)
    mesh = plsc.VectorSubcoreMesh(core_axis_name="c", subcore_axis_name="s")

    assert n_chunks >= 2 * GROUP
    n_full = n_chunks // GROUP
    n_tail = n_chunks - n_full * GROUP
    if n_tail == 0:
        n_tail = GROUP
        n_full -= 1

    @functools.partial(
        pl.kernel,
        out_type=jax.ShapeDtypeStruct((NC, n, d), jnp.float32),
        mesh=mesh,
        scratch_types=[
            pltpu.VMEM((NEB, CHUNK), jnp.int32),
            pltpu.VMEM((NEB, CHUNK), jnp.int32),
            pltpu.VMEM((NEB, CHUNK), jnp.float32),
            pltpu.VMEM((NROW, CHUNK, d), jnp.float32),
            pltpu.VMEM_SHARED((n, d), jnp.float32),
            pltpu.SemaphoreType.DMA((NEB,)),
            pltpu.SemaphoreType.DMA((NEB,)),
            pltpu.SemaphoreType.DMA((NEB,)),
            pltpu.SemaphoreType.DMA((NROW,)),
            pltpu.SemaphoreType.DMA((NROW,)),
        ],
    )
    def k(x_hbm, ep_hbm, ev_hbm, out_hbm,
          sb_v, db_v, vb_v, rows_v, acc, s_sem, d_sem, v_sem, gsem, ssem):
        cid = lax.axis_index("c")
        sid = lax.axis_index("s")
        wid = cid * NS + sid

        def eb_start(c, s):
            m = s % NEB
            pltpu.async_copy(ep_hbm.at[1, wid, c], sb_v.at[m], s_sem.at[m])
            pltpu.async_copy(ep_hbm.at[0, wid, c], db_v.at[m], d_sem.at[m])
            pltpu.async_copy(ev_hbm.at[wid, c], vb_v.at[m], v_sem.at[m])

        def eb_wait(c, s):
            m = s % NEB
            pltpu.make_async_copy(ep_hbm.at[1, wid, c], sb_v.at[m],
                                  s_sem.at[m]).wait()
            pltpu.make_async_copy(ep_hbm.at[0, wid, c], db_v.at[m],
                                  d_sem.at[m]).wait()
            pltpu.make_async_copy(ev_hbm.at[wid, c], vb_v.at[m],
                                  v_sem.at[m]).wait()

        def gather_start(c, s):
            m, r = s % NEB, s % NROW
            pltpu.async_copy(x_hbm.at[sb_v.at[m]], rows_v.at[r], gsem.at[r])

        def gather_wait(c, s):
            m, r = s % NEB, s % NROW
            pltpu.make_async_copy(x_hbm.at[sb_v.at[m]], rows_v.at[r],
                                  gsem.at[r]).wait()

        def scat_start(c, s):
            m, r = s % NEB, s % NROW
            pltpu.async_copy(rows_v.at[r], acc.at[db_v.at[m]], ssem.at[r],
                             add=True)

        def scat_wait(c, s):
            m, r = s % NEB, s % NROW
            pltpu.make_async_copy(rows_v.at[r], acc.at[db_v.at[m]],
                                  ssem.at[r]).wait()

        def scale(s):
            m, r = s % NEB, s % NROW

            def scale_body(i16, carry2):
                vvec = vb_v[m, pl.ds(i16 * LANES, LANES)]
                for ii in range(LANES):
                    v = vvec[ii]
                    row = i16 * LANES + ii
                    for j in range(d // LANES):
                        sl = (r, row, pl.ds(j * LANES, LANES))
                        rows_v[sl] = rows_v[sl] * v
                return carry2

            lax.fori_loop(0, CHUNK // LANES, scale_body, 0)

        def steady(c, s, first=False, do_g2=True, do_e4=True):
            gather_wait(c, s)
            scale(s)
            scat_start(c, s)
            if not first:
                scat_wait(c - 1, s - 1)
            if do_g2:
                eb_wait(c + 2, s + 2)
                gather_start(c + 2, s + 2)
            if do_e4:
                eb_start(c + 4, s + 4)

        for c in range(4):
            eb_start(c, c)
        for c in range(2):
            eb_wait(c, c)
            gather_start(c, c)

        zvec = jnp.zeros((LANES,), jnp.float32)

        def zero_body(i, carry):
            for j in range(d // LANES):
                rows_v[2, i, pl.ds(j * LANES, LANES)] = zvec
            return carry

        lax.fori_loop(0, CHUNK, zero_body, 0)
        r0 = sid * rows_per_tile
        for q in range(n_zfull):
            pltpu.sync_copy(rows_v.at[2],
                            acc.at[pl.ds(r0 + q * CHUNK, CHUNK)])
        if n_zrem:
            pltpu.sync_copy(rows_v.at[2, pl.ds(0, n_zrem)],
                            acc.at[pl.ds(r0 + n_zfull * CHUNK, n_zrem)])
        if rem_rows:
            @pl.when(sid == NS - 1)
            def _():
                pltpu.sync_copy(rows_v.at[2, pl.ds(0, rem_rows)],
                                acc.at[pl.ds(rem_base, rem_rows)])
        plsc.subcore_barrier()

        steady(0, 0, first=True)
        for c in range(1, GROUP):
            steady(c, c)

        def group_body(g, carry):
            c0 = g * GROUP
            for kk in range(GROUP):
                steady(c0 + kk, kk)
            return carry

        lax.fori_loop(1, n_full, group_body, 0)

        cl = n_full * GROUP
        for kk in range(n_tail):
            steady(cl + kk, kk, do_g2=(kk < n_tail - 2),
                   do_e4=(kk < n_tail - 4))
        scat_wait(n_chunks - 1, n_tail - 1)
        plsc.subcore_barrier()

        pltpu.sync_copy(acc.at[pl.ds(r0, rows_per_tile)],
                        out_hbm.at[cid, pl.ds(r0, rows_per_tile)])
        if rem_rows:
            @pl.when(sid == NS - 1)
            def _():
                pltpu.sync_copy(acc.at[pl.ds(rem_base, rem_rows)],
                                out_hbm.at[cid, pl.ds(rem_base, rem_rows)])

    return k(x, epk, evp)


def _epilogue(parts, x, x0, alpha_col, n, d, block):
    def body(parts_ref, x_ref, x0_ref, a_ref, o_ref):
        ax = parts_ref[0] + parts_ref[1]
        alph = jax.nn.sigmoid(a_ref[...].astype(jnp.float32))
        o_ref[...] = alph * 0.5 * (ax - x_ref[...]) + x0_ref[...]

    return pl.pallas_call(
        body,
        out_shape=jax.ShapeDtypeStruct((n, d), jnp.float32),
        grid=(n // block,),
        in_specs=[
            pl.BlockSpec((NC, block, d), lambda i: (0, i, 0)),
            pl.BlockSpec((block, d), lambda i: (i, 0)),
            pl.BlockSpec((block, d), lambda i: (i, 0)),
            pl.BlockSpec((block, 1), lambda i: (i, 0)),
        ],
        out_specs=pl.BlockSpec((block, d), lambda i: (i, 0)),
    )(parts, x, x0, alpha_col)


def kernel(t, x, x0, alpha_train, adj_vals, edge_index):
    n, d = x.shape
    e = edge_index.shape[1]
    n_chunks = -(-e // (NW * CHUNK))
    e_pad = NW * CHUNK * n_chunks
    pad = e_pad - e

    spread = jnp.arange(pad, dtype=edge_index.dtype) % n
    epk = jnp.concatenate(
        [edge_index, jnp.broadcast_to(spread, (2, pad))],
        axis=1).reshape(2, NW, n_chunks, CHUNK)
    evp = jnp.pad(adj_vals, (0, pad)).reshape(NW, n_chunks, CHUNK)

    parts = _sc_spmm(x, epk, evp, n_chunks, n, d)
    alpha_col = alpha_train.astype(jnp.bfloat16)[:, None]
    return _epilogue(parts, x, x0, alpha_col, n, d, 2000)
